# Initial kernel scaffold; baseline (speedup 1.0000x reference)
import jax
import jax.numpy as jnp
from jax.experimental import pallas as pl


def kernel(scaled_atomic_energy, batch, modal_type, atom_type, shift, scale):
    raise NotImplementedError("write your pallas kernel here")


# speedup vs baseline: 72.2398x; 72.2398x over previous
import functools

import jax
import jax.numpy as jnp
from jax import lax
from jax.experimental import pallas as pl
from jax.experimental.pallas import tpu as pltpu
from jax.experimental.pallas import tpu_sc as plsc

N = 100000
M = 4
T = 16
NC = 2
NS = 16
NW = NC * NS
LANES = 16
CHUNK = 3136


def _body(x_hbm, b_hbm, at_hbm, mt_hbm, tab_hbm, out_hbm,
          x_v, b_v, at_v, mt_v, tab_v, out_v):
    wid = lax.axis_index("s") * NC + lax.axis_index("c")
    base = jnp.minimum(wid * CHUNK, N - CHUNK)

    pltpu.sync_copy(mt_hbm, mt_v)
    pltpu.sync_copy(tab_hbm, tab_v)
    pltpu.sync_copy(x_hbm.at[pl.ds(base, CHUNK)], x_v)
    pltpu.sync_copy(b_hbm.at[pl.ds(base, CHUNK)], b_v)
    pltpu.sync_copy(at_hbm.at[pl.ds(base, CHUNK)], at_v)

    def step(j, carry):
        off = pl.multiple_of(j * LANES, LANES)
        b16 = b_v[pl.ds(off, LANES)]
        m16 = plsc.load_gather(mt_v, [b16])
        idx = m16 * T + at_v[pl.ds(off, LANES)]
        c16 = plsc.load_gather(tab_v, [idx])
        s16 = plsc.load_gather(tab_v, [idx + (M * T)])
        out_v[pl.ds(off, LANES)] = x_v[pl.ds(off, LANES)] * c16 + s16
        return carry

    lax.fori_loop(0, CHUNK // LANES, step, 0)
    pltpu.sync_copy(out_v, out_hbm.at[pl.ds(base, CHUNK)])


@jax.jit
def _rescale(x, b, at, mt, tab):
    mesh = plsc.VectorSubcoreMesh(core_axis_name="c", subcore_axis_name="s")
    return pl.kernel(
        _body,
        out_type=jax.ShapeDtypeStruct((N,), jnp.float32),
        mesh=mesh,
        compiler_params=pltpu.CompilerParams(needs_layout_passes=False),
        scratch_types=[
            pltpu.VMEM((CHUNK,), jnp.float32),
            pltpu.VMEM((CHUNK,), jnp.int32),
            pltpu.VMEM((CHUNK,), jnp.int32),
            pltpu.VMEM((mt.shape[0],), jnp.int32),
            pltpu.VMEM((2 * M * T,), jnp.float32),
            pltpu.VMEM((CHUNK,), jnp.float32),
        ],
    )(x, b, at, mt, tab)


def kernel(scaled_atomic_energy, batch, modal_type, atom_type, shift, scale):
    x = scaled_atomic_energy.reshape(N)
    b = batch.astype(jnp.int32)
    at = atom_type.astype(jnp.int32)
    mt = modal_type.astype(jnp.int32)
    tab = jnp.concatenate([scale.reshape(M * T), shift.reshape(M * T)])
    out = _rescale(x, b, at, mt, tab)
    return out.reshape(N, 1)


# speedup vs baseline: 83.9254x; 1.1618x over previous
import functools

import jax
import jax.numpy as jnp
from jax import lax
from jax.experimental import pallas as pl
from jax.experimental.pallas import tpu as pltpu
from jax.experimental.pallas import tpu_sc as plsc

N = 100000
M = 4
T = 16
NC = 2
NS = 16
NW = NC * NS
LANES = 16
CHUNK = 3136


def _body(x_hbm, b_hbm, at_hbm, mt_hbm, tab_hbm, out_hbm,
          x_v, b_v, at_v, mt_v, tab_v, out_v, sem):
    wid = lax.axis_index("s") * NC + lax.axis_index("c")
    base = jnp.minimum(wid * CHUNK, N - CHUNK)

    copies = [
        pltpu.async_copy(mt_hbm, mt_v, sem),
        pltpu.async_copy(tab_hbm, tab_v, sem),
        pltpu.async_copy(x_hbm.at[pl.ds(base, CHUNK)], x_v, sem),
        pltpu.async_copy(b_hbm.at[pl.ds(base, CHUNK)], b_v, sem),
        pltpu.async_copy(at_hbm.at[pl.ds(base, CHUNK)], at_v, sem),
    ]
    for c in copies:
        c.wait()

    @plsc.parallel_loop(0, CHUNK, step=LANES, unroll=4)
    def _(off):
        b16 = b_v[pl.ds(off, LANES)]
        m16 = plsc.load_gather(mt_v, [b16])
        idx = m16 * T + at_v[pl.ds(off, LANES)]
        c16 = plsc.load_gather(tab_v, [idx])
        s16 = plsc.load_gather(tab_v, [idx + (M * T)])
        out_v[pl.ds(off, LANES)] = x_v[pl.ds(off, LANES)] * c16 + s16

    pltpu.sync_copy(out_v, out_hbm.at[pl.ds(base, CHUNK)])


@jax.jit
def _rescale(x, b, at, mt, tab):
    mesh = plsc.VectorSubcoreMesh(core_axis_name="c", subcore_axis_name="s")
    return pl.kernel(
        _body,
        out_type=jax.ShapeDtypeStruct((N,), jnp.float32),
        mesh=mesh,
        compiler_params=pltpu.CompilerParams(needs_layout_passes=False),
        scratch_types=[
            pltpu.VMEM((CHUNK,), jnp.float32),
            pltpu.VMEM((CHUNK,), jnp.int32),
            pltpu.VMEM((CHUNK,), jnp.int32),
            pltpu.VMEM((mt.shape[0],), jnp.int32),
            pltpu.VMEM((2 * M * T,), jnp.float32),
            pltpu.VMEM((CHUNK,), jnp.float32),
            pltpu.SemaphoreType.DMA,
        ],
    )(x, b, at, mt, tab)


def kernel(scaled_atomic_energy, batch, modal_type, atom_type, shift, scale):
    x = scaled_atomic_energy.reshape(N)
    b = batch.astype(jnp.int32)
    at = atom_type.astype(jnp.int32)
    mt = modal_type.astype(jnp.int32)
    tab = jnp.concatenate([scale.reshape(M * T), shift.reshape(M * T)])
    out = _rescale(x, b, at, mt, tab)
    return out.reshape(N, 1)


# speedup vs baseline: 88.9018x; 1.0593x over previous
import jax
import jax.numpy as jnp
from jax import lax
from jax.experimental import pallas as pl
from jax.experimental.pallas import tpu as pltpu
from jax.experimental.pallas import tpu_sc as plsc

N = 100000
M = 4
T = 16
NC = 2
NS = 16
NW = NC * NS
LANES = 16
CHUNK = 3136


def _body(x_hbm, b_hbm, at_hbm, mt_hbm, sh_hbm, sc_hbm, out_hbm,
          x_v, b_v, at_v, mt_v, sh_v, sc_v, out_v, sem):
    wid = lax.axis_index("s") * NC + lax.axis_index("c")
    base = jnp.minimum(wid * CHUNK, N - CHUNK)

    copies = [
        pltpu.async_copy(mt_hbm, mt_v, sem),
        pltpu.async_copy(sh_hbm, sh_v, sem),
        pltpu.async_copy(sc_hbm, sc_v, sem),
        pltpu.async_copy(x_hbm.at[pl.ds(base, CHUNK)], x_v, sem),
        pltpu.async_copy(b_hbm.at[pl.ds(base, CHUNK)], b_v, sem),
        pltpu.async_copy(at_hbm.at[pl.ds(base, CHUNK)], at_v, sem),
    ]
    for c in copies:
        c.wait()

    @plsc.parallel_loop(0, CHUNK, step=LANES, unroll=4)
    def _(off):
        b16 = b_v[pl.ds(off, LANES)]
        m16 = plsc.load_gather(mt_v, [b16])
        t16 = at_v[pl.ds(off, LANES)]
        c16 = plsc.load_gather(sc_v, [m16, t16])
        s16 = plsc.load_gather(sh_v, [m16, t16])
        out_v[pl.ds(off, LANES)] = x_v[pl.ds(off, LANES)] * c16 + s16

    pltpu.sync_copy(out_v, out_hbm.at[pl.ds(base, CHUNK)])


@jax.jit
def _rescale(x, b, at, mt, sh, sc):
    mesh = plsc.VectorSubcoreMesh(core_axis_name="c", subcore_axis_name="s")
    return pl.kernel(
        _body,
        out_type=jax.ShapeDtypeStruct((N,), jnp.float32),
        mesh=mesh,
        compiler_params=pltpu.CompilerParams(needs_layout_passes=False),
        scratch_types=[
            pltpu.VMEM((CHUNK,), jnp.float32),
            pltpu.VMEM((CHUNK,), jnp.int32),
            pltpu.VMEM((CHUNK,), jnp.int32),
            pltpu.VMEM((mt.shape[0],), jnp.int32),
            pltpu.VMEM((M, T), jnp.float32),
            pltpu.VMEM((M, T), jnp.float32),
            pltpu.VMEM((CHUNK,), jnp.float32),
            pltpu.SemaphoreType.DMA,
        ],
    )(x, b, at, mt, sh, sc)


def kernel(scaled_atomic_energy, batch, modal_type, atom_type, shift, scale):
    x = lax.squeeze(scaled_atomic_energy, (1,))
    b = batch.astype(jnp.int32)
    at = atom_type.astype(jnp.int32)
    mt = modal_type.astype(jnp.int32)
    out = _rescale(x, b, at, mt, shift, scale)
    return out[:, None]
